# baseline (speedup 1.0000x reference)
import functools

import jax
import jax.numpy as jnp
from jax import lax
from jax.experimental import pallas as pl
from jax.experimental.pallas import tpu as pltpu
from jax.experimental.pallas import tpu_sc as plsc

N_USERS = 25000
N_NODES_TOTAL = 50000
DIM = 64
N_EDGES_REAL = 800000
CH = 128
EPT = 50432
NCH = EPT // CH
EDGES_PAD = 16 * EPT
CEPT = EPT + 12 * CH
NHALF = 25000
ACC_ROWS = 25600
ZROWS = 1600
OUT_FULL = 1568
OUT_LAST = 25000 - 15 * OUT_FULL

_mesh = plsc.VectorSubcoreMesh(core_axis_name="c", subcore_axis_name="s")
_sc_params = pltpu.CompilerParams(use_tc_tiling_on_sc=False)
_sc_params_nl = pltpu.CompilerParams(use_tc_tiling_on_sc=False, needs_layout_passes=False)


@functools.partial(
    pl.kernel,
    mesh=_mesh,
    compiler_params=_sc_params_nl,
    out_type=(
        jax.ShapeDtypeStruct((32 * CEPT,), jnp.int32),
        jax.ShapeDtypeStruct((32 * CEPT,), jnp.int32),
        jax.ShapeDtypeStruct((32 * CEPT,), jnp.float32),
        jax.ShapeDtypeStruct((2, 16, 16), jnp.int32),
    ),
    scratch_types=[
        pltpu.VMEM((2, CH), jnp.int32),
        pltpu.VMEM((2, CH), jnp.int32),
        pltpu.VMEM((2, CH), jnp.float32),
        pltpu.VMEM((272,), jnp.int32),
        pltpu.VMEM((272,), jnp.int32),
        pltpu.VMEM((272,), jnp.float32),
        pltpu.VMEM((16,), jnp.int32),
        pltpu.SemaphoreType.DMA,
    ],
)
def _partition(src, dst, vals, csrc, cdst, cvals, counts,
               sv, dv, vv, st_s, st_d, st_v, cnt_v, sem):
    c = lax.axis_index("c")
    s = lax.axis_index("s")
    base = c * NHALF
    ebase = s * EPT
    cbase = (c * 16 + s) * CEPT

    def issue(ci, b):
        off = ebase + ci * CH
        pltpu.async_copy(src.at[pl.ds(off, CH)], sv.at[b], sem)
        pltpu.async_copy(dst.at[pl.ds(off, CH)], dv.at[b], sem)
        pltpu.async_copy(vals.at[pl.ds(off, CH)], vv.at[b], sem)

    def wait(b):
        pltpu.make_async_copy(src.at[pl.ds(0, CH)], sv.at[b], sem).wait()
        pltpu.make_async_copy(dst.at[pl.ds(0, CH)], dv.at[b], sem).wait()
        pltpu.make_async_copy(vals.at[pl.ds(0, CH)], vv.at[b], sem).wait()

    def flush(blocks):
        boff = cbase + blocks * CH
        pltpu.sync_copy(st_s.at[pl.ds(0, CH)], csrc.at[pl.ds(boff, CH)])
        pltpu.sync_copy(st_d.at[pl.ds(0, CH)], cdst.at[pl.ds(boff, CH)])
        pltpu.sync_copy(st_v.at[pl.ds(0, CH)], cvals.at[pl.ds(boff, CH)])

    zi16 = jnp.zeros((16,), jnp.int32)
    zf16 = jnp.zeros((16,), jnp.float32)

    def zero_stage():
        for j in range(CH // 16):
            sl = pl.ds(16 * j, 16)
            st_s[sl] = zi16
            st_d[sl] = zi16
            st_v[sl] = zf16

    issue(0, 0)

    def pair(g2, st):
        for b in range(2):
            ci = 2 * g2 + b
            wait(b)

            @pl.when(ci + 1 < NCH)
            def _():
                issue(ci + 1, 1 - b)

            p, blocks = st
            for grp in range(CH // 16):
                sl = pl.ds(grp * 16, 16)
                loc = dv[b, sl] - base
                m = (loc >= 0) & (loc < NHALF)
                pos = plsc.cumsum(m.astype(jnp.int32)) - 1 + p
                plsc.store_scatter(st_s, [pos], sv[b, sl], mask=m)
                plsc.store_scatter(st_d, [pos], loc, mask=m)
                plsc.store_scatter(st_v, [pos], vv[b, sl], mask=m)
                p = p + plsc.all_reduce_population_count(m)[0]

            fl = p >= CH

            @pl.when(fl)
            def _():
                flush(blocks)
                for j in range(CH // 16):
                    sl_lo = pl.ds(16 * j, 16)
                    sl_hi = pl.ds(CH + 16 * j, 16)
                    st_s[sl_lo] = st_s[sl_hi]
                    st_d[sl_lo] = st_d[sl_hi]
                    st_v[sl_lo] = st_v[sl_hi]

            blocks = jnp.where(fl, blocks + 1, blocks)
            p = jnp.where(fl, p - CH, p)
            st = (p, blocks)
        return st

    p, blocks = lax.fori_loop(
        0, NCH // 2, pair, (jnp.int32(0), jnp.int32(0))
    )

    tail = jnp.arange(16, dtype=jnp.int32) + p
    plsc.store_scatter(st_s, [tail], zi16)
    plsc.store_scatter(st_d, [tail], zi16)
    plsc.store_scatter(st_v, [tail], zf16)
    for j in range(CH // 16):
        @pl.when(jnp.int32(16 * j) >= p)
        def _():
            sl = pl.ds(16 * j, 16)
            st_s[sl] = zi16
            st_d[sl] = zi16
            st_v[sl] = zf16

    @pl.when(p > 0)
    def _():
        flush(blocks)

    n0 = blocks + jnp.where(p > 0, 1, 0).astype(jnp.int32)
    zero_stage()
    for k in range(12):
        flush(n0 + k)
    nch12 = jnp.maximum((n0 + 11) // 12, 1)
    cnt_v[...] = jnp.full((16,), 1, jnp.int32) * nch12
    pltpu.sync_copy(cnt_v, counts.at[c, s])


@functools.partial(
    pl.kernel,
    mesh=_mesh,
    compiler_params=_sc_params,
    out_type=jax.ShapeDtypeStruct((N_NODES_TOTAL, DIM), jnp.float32),
    scratch_types=[
        pltpu.VMEM((4, CH), jnp.int32),
        pltpu.VMEM((4, CH), jnp.int32),
        pltpu.VMEM((4, CH + 16), jnp.float32),
        pltpu.VMEM((3, CH, DIM), jnp.float32),
        pltpu.VMEM((16,), jnp.int32),
        pltpu.VMEM_SHARED((ACC_ROWS, DIM), jnp.float32),
        pltpu.SemaphoreType.DMA,
        pltpu.SemaphoreType.DMA,
        pltpu.SemaphoreType.DMA,
    ],
)
def _layer(tab, csrc, cdst, cvals, counts, zrows, out,
           src_v, dst_v, vals_v, rows_v, cnt_v, acc, sem_l, sem_g, sem_s):
    c = lax.axis_index("c")
    s = lax.axis_index("s")
    base = c * NHALF
    cbase = (c * 16 + s) * CEPT

    pltpu.sync_copy(counts.at[c, s], cnt_v)
    nch12 = cnt_v[...][0]
    nch = nch12 * 12

    pltpu.sync_copy(zrows, acc.at[pl.ds(s * ZROWS, ZROWS)])
    plsc.subcore_barrier()

    def issue_linear(ci, b):
        off = cbase + ci * CH
        pltpu.async_copy(csrc.at[pl.ds(off, CH)], src_v.at[b], sem_l)
        pltpu.async_copy(cdst.at[pl.ds(off, CH)], dst_v.at[b], sem_l)
        pltpu.async_copy(cvals.at[pl.ds(off, CH)], vals_v.at[b, pl.ds(0, CH)], sem_l)

    def wait_linear(b):
        pltpu.make_async_copy(csrc.at[pl.ds(0, CH)], src_v.at[b], sem_l).wait()
        pltpu.make_async_copy(cdst.at[pl.ds(0, CH)], dst_v.at[b], sem_l).wait()
        pltpu.make_async_copy(
            cvals.at[pl.ds(0, CH)], vals_v.at[b, pl.ds(0, CH)], sem_l
        ).wait()

    def issue_gather(ci_rb, ib):
        pltpu.async_copy(tab.at[src_v.at[ib]], rows_v.at[ci_rb], sem_g)

    def wait_gather(ci_rb, ib):
        pltpu.make_async_copy(tab.at[src_v.at[ib]], rows_v.at[ci_rb], sem_g).wait()

    def process(rb, ib):
        def scale(k, cc):
            v = vals_v[ib, pl.ds(k, 16)][0]
            for j in range(DIM // 16):
                sl = pl.ds(j * 16, 16)
                rows_v[rb, k, sl] = rows_v[rb, k, sl] * v
            return cc

        lax.fori_loop(0, CH, scale, 0, unroll=8)

    def issue_scatter(rb, ib):
        pltpu.async_copy(rows_v.at[rb], acc.at[dst_v.at[ib]], sem_s, add=True)

    def wait_scatter(rb, ib):
        pltpu.make_async_copy(rows_v.at[rb], acc.at[dst_v.at[ib]], sem_s).wait()

    issue_linear(0, 0)
    wait_linear(0)
    issue_gather(0, 0)
    issue_linear(1, 1)
    wait_linear(1)
    issue_gather(1, 1)
    issue_linear(2, 2)

    def twelve(g, carry):
        for k in range(12):
            ci = 12 * g + k
            rb = k % 3
            ib = k % 4

            if k == 0:
                @pl.when(g > 0)
                def _():
                    wait_scatter(2, 3)
            else:
                wait_scatter((k - 1) % 3, (k - 1) % 4)

            @pl.when(ci + 3 < nch)
            def _():
                issue_linear(ci + 3, (ib + 3) % 4)

            @pl.when(ci + 2 < nch)
            def _():
                wait_linear((ib + 2) % 4)
                issue_gather((rb + 2) % 3, (ib + 2) % 4)

            wait_gather(rb, ib)
            process(rb, ib)
            issue_scatter(rb, ib)

        return carry

    lax.fori_loop(0, nch12, twelve, 0)
    wait_scatter(2, 3)
    plsc.subcore_barrier()

    @pl.when(s < 15)
    def _copy_full():
        r0 = s * OUT_FULL
        pltpu.sync_copy(
            acc.at[pl.ds(r0, OUT_FULL)], out.at[pl.ds(base + r0, OUT_FULL)]
        )

    @pl.when(s == 15)
    def _copy_last():
        r0 = 15 * OUT_FULL
        pltpu.sync_copy(
            acc.at[pl.ds(r0, OUT_LAST)], out.at[pl.ds(base + r0, OUT_LAST)]
        )


QTOT = 16384
QPT = QTOT // 32
QCH = 128


@functools.partial(
    pl.kernel,
    mesh=_mesh,
    compiler_params=_sc_params,
    out_type=(
        jax.ShapeDtypeStruct((QTOT, DIM), jnp.float32),
        jax.ShapeDtypeStruct((QTOT, DIM), jnp.float32),
    ),
    scratch_types=[
        pltpu.VMEM((QCH,), jnp.int32),
        pltpu.VMEM((QCH, DIM), jnp.float32),
        pltpu.VMEM((QCH, DIM), jnp.float32),
        pltpu.SemaphoreType.DMA,
    ],
)
def _qsum(t0, t1, t2, t3, uq, iq, usum, isum, q_v, acc_v, tmp, sem):
    c = lax.axis_index("c")
    s = lax.axis_index("s")
    wid = s * 2 + c
    qbase = wid * QPT

    def accum_tmp():
        def add_row(r, cc):
            for j in range(DIM // 16):
                sl = pl.ds(j * 16, 16)
                acc_v[r, sl] = acc_v[r, sl] + tmp[r, sl]
            return cc

        lax.fori_loop(0, QCH, add_row, 0)

    def side(q_hbm, out_hbm, ci):
        off = qbase + ci * QCH
        pltpu.sync_copy(q_hbm.at[pl.ds(off, QCH)], q_v)
        pltpu.async_copy(t0.at[q_v], acc_v, sem).wait()
        for t in (t1, t2, t3):
            pltpu.async_copy(t.at[q_v], tmp, sem).wait()
            accum_tmp()
        pltpu.sync_copy(acc_v, out_hbm.at[pl.ds(off, QCH)])

    def chunk(ci, carry):
        side(uq, usum, ci)
        side(iq, isum, ci)
        return carry

    lax.fori_loop(0, QPT // QCH, chunk, 0)


def _dot_body(u_ref, i_ref, o_ref):
    o_ref[...] = jnp.sum(u_ref[...] * i_ref[...], axis=1) * jnp.float32(1.0 / 16.0)


def _mean_dot(usum, isum):
    return pl.pallas_call(
        _dot_body,
        out_shape=jax.ShapeDtypeStruct((QTOT,), jnp.float32),
        grid=(16,),
        in_specs=[
            pl.BlockSpec((QTOT // 16, DIM), lambda i: (i, 0)),
            pl.BlockSpec((QTOT // 16, DIM), lambda i: (i, 0)),
        ],
        out_specs=pl.BlockSpec((QTOT // 16,), lambda i: (i,)),
    )(usum, isum)


def kernel(user_emb, item_emb, edge_index, edge_vals, users, items):
    t0 = jnp.concatenate([user_emb, item_emb], axis=0)
    src = edge_index[0].astype(jnp.int32)
    dst = edge_index[1].astype(jnp.int32)
    padn = EDGES_PAD - N_EDGES_REAL
    zi = jnp.zeros((padn,), jnp.int32)
    src = jnp.concatenate([src, zi])
    dst = jnp.concatenate([dst, zi])
    vals = jnp.concatenate(
        [edge_vals.astype(jnp.float32), jnp.zeros((padn,), jnp.float32)]
    )
    zrows = jnp.zeros((ZROWS, DIM), jnp.float32)
    csrc, cdst, cvals, counts = _partition(src, dst, vals)
    t1 = _layer(t0, csrc, cdst, cvals, counts, zrows)
    t2 = _layer(t1, csrc, cdst, cvals, counts, zrows)
    t3 = _layer(t2, csrc, cdst, cvals, counts, zrows)
    uq = users.astype(jnp.int32)
    iq = items.astype(jnp.int32) + jnp.int32(N_USERS)
    usum, isum = _qsum(t0, t1, t2, t3, uq, iq)
    return _mean_dot(usum, isum)


# speedup vs baseline: 1.6640x; 1.6640x over previous
import functools

import jax
import jax.numpy as jnp
from jax import lax
from jax.experimental import pallas as pl
from jax.experimental.pallas import tpu as pltpu
from jax.experimental.pallas import tpu_sc as plsc

N_USERS = 25000
N_NODES_TOTAL = 50000
DIM = 64
N_EDGES_REAL = 800000
CH = 96
EPT = 50496
NCH = EPT // CH
EDGES_PAD = 16 * EPT
CEPT = EPT + 4 * CH
NHALF = 25000
ACC_ROWS = 25600
ZROWS = 1600
OUT_FULL = 1568
OUT_LAST = 25000 - 15 * OUT_FULL

_mesh = plsc.VectorSubcoreMesh(core_axis_name="c", subcore_axis_name="s")
_sc_params = pltpu.CompilerParams(use_tc_tiling_on_sc=False)
_sc_params_nl = pltpu.CompilerParams(use_tc_tiling_on_sc=False, needs_layout_passes=False)


@functools.partial(
    pl.kernel,
    mesh=_mesh,
    compiler_params=_sc_params_nl,
    out_type=(
        jax.ShapeDtypeStruct((32 * CEPT,), jnp.int32),
        jax.ShapeDtypeStruct((32 * CEPT,), jnp.int32),
        jax.ShapeDtypeStruct((32 * CEPT,), jnp.float32),
        jax.ShapeDtypeStruct((2, 16, 16), jnp.int32),
    ),
    scratch_types=[
        pltpu.VMEM((2, CH), jnp.int32),
        pltpu.VMEM((2, CH), jnp.int32),
        pltpu.VMEM((2, CH), jnp.float32),
        pltpu.VMEM((208,), jnp.int32),
        pltpu.VMEM((208,), jnp.int32),
        pltpu.VMEM((208,), jnp.float32),
        pltpu.VMEM((16,), jnp.int32),
        pltpu.SemaphoreType.DMA,
    ],
)
def _partition(src, dst, vals, csrc, cdst, cvals, counts,
               sv, dv, vv, st_s, st_d, st_v, cnt_v, sem):
    c = lax.axis_index("c")
    s = lax.axis_index("s")
    base = c * NHALF
    ebase = s * EPT
    cbase = (c * 16 + s) * CEPT

    def issue(ci, b):
        off = ebase + ci * CH
        pltpu.async_copy(src.at[pl.ds(off, CH)], sv.at[b], sem)
        pltpu.async_copy(dst.at[pl.ds(off, CH)], dv.at[b], sem)
        pltpu.async_copy(vals.at[pl.ds(off, CH)], vv.at[b], sem)

    def wait(b):
        pltpu.make_async_copy(src.at[pl.ds(0, CH)], sv.at[b], sem).wait()
        pltpu.make_async_copy(dst.at[pl.ds(0, CH)], dv.at[b], sem).wait()
        pltpu.make_async_copy(vals.at[pl.ds(0, CH)], vv.at[b], sem).wait()

    def flush(blocks):
        boff = cbase + blocks * CH
        pltpu.sync_copy(st_s.at[pl.ds(0, CH)], csrc.at[pl.ds(boff, CH)])
        pltpu.sync_copy(st_d.at[pl.ds(0, CH)], cdst.at[pl.ds(boff, CH)])
        pltpu.sync_copy(st_v.at[pl.ds(0, CH)], cvals.at[pl.ds(boff, CH)])

    zi16 = jnp.zeros((16,), jnp.int32)
    zf16 = jnp.zeros((16,), jnp.float32)

    def zero_stage():
        for j in range(CH // 16):
            sl = pl.ds(16 * j, 16)
            st_s[sl] = zi16
            st_d[sl] = zi16
            st_v[sl] = zf16

    issue(0, 0)

    def pair(g2, st):
        for b in range(2):
            ci = 2 * g2 + b
            wait(b)

            @pl.when(ci + 1 < NCH)
            def _():
                issue(ci + 1, 1 - b)

            p, blocks = st
            for grp in range(CH // 16):
                sl = pl.ds(grp * 16, 16)
                loc = dv[b, sl] - base
                m = (loc >= 0) & (loc < NHALF)
                pos = plsc.cumsum(m.astype(jnp.int32)) - 1 + p
                plsc.store_scatter(st_s, [pos], sv[b, sl], mask=m)
                plsc.store_scatter(st_d, [pos], loc, mask=m)
                plsc.store_scatter(st_v, [pos], vv[b, sl], mask=m)
                p = p + plsc.all_reduce_population_count(m)[0]

            fl = p >= CH

            @pl.when(fl)
            def _():
                flush(blocks)
                for j in range(CH // 16):
                    sl_lo = pl.ds(16 * j, 16)
                    sl_hi = pl.ds(CH + 16 * j, 16)
                    st_s[sl_lo] = st_s[sl_hi]
                    st_d[sl_lo] = st_d[sl_hi]
                    st_v[sl_lo] = st_v[sl_hi]

            blocks = jnp.where(fl, blocks + 1, blocks)
            p = jnp.where(fl, p - CH, p)
            st = (p, blocks)
        return st

    p, blocks = lax.fori_loop(
        0, NCH // 2, pair, (jnp.int32(0), jnp.int32(0))
    )

    tail = jnp.arange(16, dtype=jnp.int32) + p
    plsc.store_scatter(st_s, [tail], zi16)
    plsc.store_scatter(st_d, [tail], zi16)
    plsc.store_scatter(st_v, [tail], zf16)
    for j in range(CH // 16):
        @pl.when(jnp.int32(16 * j) >= p)
        def _():
            sl = pl.ds(16 * j, 16)
            st_s[sl] = zi16
            st_d[sl] = zi16
            st_v[sl] = zf16

    @pl.when(p > 0)
    def _():
        flush(blocks)

    n0 = blocks + jnp.where(p > 0, 1, 0).astype(jnp.int32)
    zero_stage()
    for k in range(4):
        flush(n0 + k)
    nch4 = jnp.maximum((n0 + 3) // 4, 1)
    cnt_v[...] = jnp.full((16,), 1, jnp.int32) * nch4
    pltpu.sync_copy(cnt_v, counts.at[c, s])


@functools.partial(
    pl.kernel,
    mesh=_mesh,
    compiler_params=_sc_params,
    out_type=jax.ShapeDtypeStruct((N_NODES_TOTAL, DIM), jnp.float32),
    scratch_types=[
        pltpu.VMEM((4, CH), jnp.int32),
        pltpu.VMEM((4, CH), jnp.int32),
        pltpu.VMEM((4, CH + 16), jnp.float32),
        pltpu.VMEM((4, CH, DIM), jnp.float32),
        pltpu.VMEM((16,), jnp.int32),
        pltpu.VMEM_SHARED((ACC_ROWS, DIM), jnp.float32),
        pltpu.SemaphoreType.DMA,
        pltpu.SemaphoreType.DMA,
        pltpu.SemaphoreType.DMA,
    ],
)
def _layer(tab, csrc, cdst, cvals, counts, zrows, out,
           src_v, dst_v, vals_v, rows_v, cnt_v, acc, sem_l, sem_g, sem_s):
    c = lax.axis_index("c")
    s = lax.axis_index("s")
    base = c * NHALF
    cbase = (c * 16 + s) * CEPT

    pltpu.sync_copy(counts.at[c, s], cnt_v)
    nch4 = cnt_v[...][0]
    nch = nch4 * 4

    pltpu.sync_copy(zrows, acc.at[pl.ds(s * ZROWS, ZROWS)])
    plsc.subcore_barrier()

    def issue_linear(ci, b):
        off = cbase + ci * CH
        pltpu.async_copy(csrc.at[pl.ds(off, CH)], src_v.at[b], sem_l)
        pltpu.async_copy(cdst.at[pl.ds(off, CH)], dst_v.at[b], sem_l)
        pltpu.async_copy(cvals.at[pl.ds(off, CH)], vals_v.at[b, pl.ds(0, CH)], sem_l)

    def wait_linear(b):
        pltpu.make_async_copy(csrc.at[pl.ds(0, CH)], src_v.at[b], sem_l).wait()
        pltpu.make_async_copy(cdst.at[pl.ds(0, CH)], dst_v.at[b], sem_l).wait()
        pltpu.make_async_copy(
            cvals.at[pl.ds(0, CH)], vals_v.at[b, pl.ds(0, CH)], sem_l
        ).wait()

    def issue_gather(b):
        pltpu.async_copy(tab.at[src_v.at[b]], rows_v.at[b], sem_g)

    def wait_gather(b):
        pltpu.make_async_copy(tab.at[src_v.at[b]], rows_v.at[b], sem_g).wait()

    def process(b):
        def scale(k, cc):
            v = vals_v[b, pl.ds(k, 16)][0]
            for j in range(DIM // 16):
                sl = pl.ds(j * 16, 16)
                rows_v[b, k, sl] = rows_v[b, k, sl] * v
            return cc

        lax.fori_loop(0, CH, scale, 0, unroll=8)

    def issue_scatter(b):
        pltpu.async_copy(rows_v.at[b], acc.at[dst_v.at[b]], sem_s, add=True)

    def wait_scatter(b):
        pltpu.make_async_copy(rows_v.at[b], acc.at[dst_v.at[b]], sem_s).wait()

    issue_linear(0, 0)
    wait_linear(0)
    issue_gather(0)
    issue_linear(1, 1)
    wait_linear(1)
    issue_gather(1)
    issue_linear(2, 2)

    def four(g, carry):
        for b in range(4):
            ci = 4 * g + b

            if b == 0:
                @pl.when(g > 0)
                def _():
                    wait_scatter(3)
            else:
                wait_scatter(b - 1)

            @pl.when(ci + 3 < nch)
            def _():
                issue_linear(ci + 3, (b + 3) % 4)

            @pl.when(ci + 2 < nch)
            def _():
                wait_linear((b + 2) % 4)
                issue_gather((b + 2) % 4)

            wait_gather(b)
            process(b)
            issue_scatter(b)

        return carry

    lax.fori_loop(0, nch4, four, 0)
    wait_scatter(3)
    plsc.subcore_barrier()

    @pl.when(s < 15)
    def _copy_full():
        r0 = s * OUT_FULL
        pltpu.sync_copy(
            acc.at[pl.ds(r0, OUT_FULL)], out.at[pl.ds(base + r0, OUT_FULL)]
        )

    @pl.when(s == 15)
    def _copy_last():
        r0 = 15 * OUT_FULL
        pltpu.sync_copy(
            acc.at[pl.ds(r0, OUT_LAST)], out.at[pl.ds(base + r0, OUT_LAST)]
        )


QTOT = 16384
QPT = QTOT // 32
QCH = 128


@functools.partial(
    pl.kernel,
    mesh=_mesh,
    compiler_params=_sc_params,
    out_type=(
        jax.ShapeDtypeStruct((QTOT, DIM), jnp.float32),
        jax.ShapeDtypeStruct((QTOT, DIM), jnp.float32),
    ),
    scratch_types=[
        pltpu.VMEM((QCH,), jnp.int32),
        pltpu.VMEM((QCH, DIM), jnp.float32),
        pltpu.VMEM((QCH, DIM), jnp.float32),
        pltpu.SemaphoreType.DMA,
    ],
)
def _qsum(t0, t1, t2, t3, uq, iq, usum, isum, q_v, acc_v, tmp, sem):
    c = lax.axis_index("c")
    s = lax.axis_index("s")
    wid = s * 2 + c
    qbase = wid * QPT

    def accum_tmp():
        def add_row(r, cc):
            for j in range(DIM // 16):
                sl = pl.ds(j * 16, 16)
                acc_v[r, sl] = acc_v[r, sl] + tmp[r, sl]
            return cc

        lax.fori_loop(0, QCH, add_row, 0)

    def side(q_hbm, out_hbm, ci):
        off = qbase + ci * QCH
        pltpu.sync_copy(q_hbm.at[pl.ds(off, QCH)], q_v)
        pltpu.async_copy(t0.at[q_v], acc_v, sem).wait()
        for t in (t1, t2, t3):
            pltpu.async_copy(t.at[q_v], tmp, sem).wait()
            accum_tmp()
        pltpu.sync_copy(acc_v, out_hbm.at[pl.ds(off, QCH)])

    def chunk(ci, carry):
        side(uq, usum, ci)
        side(iq, isum, ci)
        return carry

    lax.fori_loop(0, QPT // QCH, chunk, 0)


def _dot_body(u_ref, i_ref, o_ref):
    o_ref[...] = jnp.sum(u_ref[...] * i_ref[...], axis=1) * jnp.float32(1.0 / 16.0)


def _mean_dot(usum, isum):
    return pl.pallas_call(
        _dot_body,
        out_shape=jax.ShapeDtypeStruct((QTOT,), jnp.float32),
        grid=(16,),
        in_specs=[
            pl.BlockSpec((QTOT // 16, DIM), lambda i: (i, 0)),
            pl.BlockSpec((QTOT // 16, DIM), lambda i: (i, 0)),
        ],
        out_specs=pl.BlockSpec((QTOT // 16,), lambda i: (i,)),
    )(usum, isum)


def kernel(user_emb, item_emb, edge_index, edge_vals, users, items):
    t0 = jnp.concatenate([user_emb, item_emb], axis=0)
    src = edge_index[0].astype(jnp.int32)
    dst = edge_index[1].astype(jnp.int32)
    padn = EDGES_PAD - N_EDGES_REAL
    zi = jnp.zeros((padn,), jnp.int32)
    src = jnp.concatenate([src, zi])
    dst = jnp.concatenate([dst, zi])
    vals = jnp.concatenate(
        [edge_vals.astype(jnp.float32), jnp.zeros((padn,), jnp.float32)]
    )
    zrows = jnp.zeros((ZROWS, DIM), jnp.float32)
    csrc, cdst, cvals, counts = _partition(src, dst, vals)
    t1 = _layer(t0, csrc, cdst, cvals, counts, zrows)
    t2 = _layer(t1, csrc, cdst, cvals, counts, zrows)
    t3 = _layer(t2, csrc, cdst, cvals, counts, zrows)
    uq = users.astype(jnp.int32)
    iq = items.astype(jnp.int32) + jnp.int32(N_USERS)
    usum, isum = _qsum(t0, t1, t2, t3, uq, iq)
    return _mean_dot(usum, isum)
